# Initial kernel scaffold; baseline (speedup 1.0000x reference)
import jax
import jax.numpy as jnp
from jax.experimental import pallas as pl


def kernel(features_1, features_2, edge_index_1, edge_index_2, avg_v, params):
    raise NotImplementedError("write your pallas kernel here")


# speedup vs baseline: 2.8350x; 2.8350x over previous
import functools

import jax
import jax.numpy as jnp
from jax import lax
from jax.experimental import pallas as pl
from jax.experimental.pallas import tpu as pltpu
from jax.experimental.pallas import tpu_sc as plsc

N_NODES = 2048
N_EDGES = 32768
R = 2 * N_NODES
E = 2 * N_EDGES
NC, NS = 2, 16
NW = NC * NS
CH = 128
E_PER_W = E // NW
N_CH = E_PER_W // CH
ROWS_PER_S = R // NS


def _sc_segment_sum(x, src, dst, zeros):
    F = x.shape[1]
    mesh = plsc.VectorSubcoreMesh(core_axis_name="c", subcore_axis_name="s")

    @functools.partial(
        pl.kernel,
        out_type=jax.ShapeDtypeStruct((NC, R, F), jnp.float32),
        mesh=mesh,
        compiler_params=pltpu.CompilerParams(use_tc_tiling_on_sc=False),
        scratch_types=[
            pltpu.VMEM((CH,), jnp.int32),
            pltpu.VMEM((CH,), jnp.int32),
            pltpu.VMEM((CH, F), jnp.float32),
            pltpu.VMEM_SHARED((R, F), jnp.float32),
            pltpu.SemaphoreType.DMA,
        ],
    )
    def k(x_hbm, src_hbm, dst_hbm, z_hbm, out_hbm, srcv, dstv, rows, acc,
          sem):
        cid = lax.axis_index("c")
        sid = lax.axis_index("s")
        wid = sid * NC + cid
        pltpu.sync_copy(z_hbm.at[pl.ds(sid * ROWS_PER_S, ROWS_PER_S)],
                        acc.at[pl.ds(sid * ROWS_PER_S, ROWS_PER_S)])
        plsc.subcore_barrier()

        base0 = wid * E_PER_W

        @pl.loop(0, N_CH)
        def _(ci):
            base = base0 + ci * CH
            pltpu.sync_copy(src_hbm.at[pl.ds(base, CH)], srcv)
            pltpu.sync_copy(dst_hbm.at[pl.ds(base, CH)], dstv)
            pltpu.async_copy(x_hbm.at[srcv], rows, sem).wait()
            pltpu.sync_copy(rows, acc.at[dstv], add=True)

        plsc.subcore_barrier()
        pltpu.sync_copy(acc.at[pl.ds(sid * ROWS_PER_S, ROWS_PER_S)],
                        out_hbm.at[cid, pl.ds(sid * ROWS_PER_S, ROWS_PER_S)])

    return k(x, src, dst, zeros)


def _gin_dense(x, agg0, agg1, scale, W1, b1, W2, b2, g, be):
    din, dh = W1.shape

    def body(x_r, a0_r, a1_r, sc_r, W1_r, b1_r, W2_r, b2_r, g_r, be_r,
             f_r, xn_r):
        h0 = x_r[...] * sc_r[0, 0] + a0_r[...] + a1_r[...]
        h = jnp.dot(h0, W1_r[...], preferred_element_type=jnp.float32)
        h = jnp.maximum(h + b1_r[...], 0.0)
        h = jnp.dot(h, W2_r[...], preferred_element_type=jnp.float32)
        h = h + b2_r[...]

        def bn(t):
            m = jnp.mean(t, axis=0, keepdims=True)
            v = jnp.mean((t - m) * (t - m), axis=0, keepdims=True)
            return (t - m) * lax.rsqrt(v + 1e-5) * g_r[...] + be_r[...]

        f = jnp.concatenate([bn(h[:N_NODES]), bn(h[N_NODES:])], axis=0)
        f_r[...] = f
        xn_r[...] = jnp.maximum(f, 0.0)

    return pl.pallas_call(
        body,
        out_shape=(
            jax.ShapeDtypeStruct((R, dh), jnp.float32),
            jax.ShapeDtypeStruct((R, dh), jnp.float32),
        ),
        in_specs=[
            pl.BlockSpec(memory_space=pltpu.VMEM),
            pl.BlockSpec(memory_space=pltpu.VMEM),
            pl.BlockSpec(memory_space=pltpu.VMEM),
            pl.BlockSpec(memory_space=pltpu.SMEM),
            pl.BlockSpec(memory_space=pltpu.VMEM),
            pl.BlockSpec(memory_space=pltpu.VMEM),
            pl.BlockSpec(memory_space=pltpu.VMEM),
            pl.BlockSpec(memory_space=pltpu.VMEM),
            pl.BlockSpec(memory_space=pltpu.VMEM),
            pl.BlockSpec(memory_space=pltpu.VMEM),
        ],
    )(x, agg0, agg1, scale, W1, b1, W2, b2, g, be)


def _mlp(x0, f1, f2, f3, W1, b1, W2, b2, W3, b3):
    BR = 1024
    d0, d1, d2, d3 = x0.shape[1], f1.shape[1], f2.shape[1], f3.shape[1]
    dh1 = W1.shape[1]
    dh2 = W2.shape[1]
    do = W3.shape[1]

    def body(x0_r, f1_r, f2_r, f3_r, W1_r, b1_r, W2_r, b2_r, W3_r, b3_r, o_r):
        W1v = W1_r[...]
        h = jnp.dot(x0_r[...], W1v[:d0], preferred_element_type=jnp.float32)
        h += jnp.dot(f1_r[...], W1v[d0:d0 + d1],
                     preferred_element_type=jnp.float32)
        h += jnp.dot(f2_r[...], W1v[d0 + d1:d0 + d1 + d2],
                     preferred_element_type=jnp.float32)
        h += jnp.dot(f3_r[...], W1v[d0 + d1 + d2:],
                     preferred_element_type=jnp.float32)
        h = jnp.maximum(h + b1_r[...], 0.0)
        h = jnp.maximum(
            jnp.dot(h, W2_r[...], preferred_element_type=jnp.float32)
            + b2_r[...], 0.0)
        o_r[...] = (jnp.dot(h, W3_r[...], preferred_element_type=jnp.float32)
                    + b3_r[...])

    full = lambda s: pl.BlockSpec(s, lambda i: (0, 0))
    return pl.pallas_call(
        body,
        grid=(R // BR,),
        out_shape=jax.ShapeDtypeStruct((R, do), jnp.float32),
        in_specs=[
            pl.BlockSpec((BR, d0), lambda i: (i, 0)),
            pl.BlockSpec((BR, d1), lambda i: (i, 0)),
            pl.BlockSpec((BR, d2), lambda i: (i, 0)),
            pl.BlockSpec((BR, d3), lambda i: (i, 0)),
            full(W1.shape), full(b1.shape), full(W2.shape), full(b2.shape),
            full(W3.shape), full(b3.shape),
        ],
        out_specs=pl.BlockSpec((BR, do), lambda i: (i, 0)),
    )(x0, f1, f2, f3, W1, b1, W2, b2, W3, b3)


def _a2(af2, Wcflat):
    def body(x_r, w_r, o_r):
        o_r[...] = jnp.dot(x_r[...], w_r[...],
                           preferred_element_type=jnp.float32)

    return pl.pallas_call(
        body,
        out_shape=jax.ShapeDtypeStruct((N_NODES, Wcflat.shape[1]),
                                       jnp.float32),
    )(af2, Wcflat)


def _cost(af1, A2, Wm1, bm1, Wm2, bm2):
    BR = 128
    KH = Wm1.shape[0]
    F3 = af1.shape[1]

    def body(x1_r, A2_r, wm1_r, bm1_r, wm2_r, bm2_r, o_r):
        x1 = x1_r[...]
        A2v = A2_r[...]
        Ms = []
        for k in range(KH):
            Ms.append(lax.dot_general(
                x1, A2v[:, k * F3:(k + 1) * F3],
                (((1,), (1,)), ((), ())),
                preferred_element_type=jnp.float32))
        cost = jnp.zeros_like(Ms[0]) + bm2_r[0]
        for m in range(KH):
            s = Ms[0] * wm1_r[0, m]
            for k in range(1, KH):
                s += Ms[k] * wm1_r[k, m]
            cost += jnp.maximum(s + bm1_r[m], 0.0) * wm2_r[m]
        o_r[...] = jnp.tanh(cost)

    return pl.pallas_call(
        body,
        grid=(N_NODES // BR,),
        out_shape=jax.ShapeDtypeStruct((N_NODES, N_NODES), jnp.float32),
        in_specs=[
            pl.BlockSpec((BR, F3), lambda i: (i, 0)),
            pl.BlockSpec(A2.shape, lambda i: (0, 0)),
            pl.BlockSpec(memory_space=pltpu.SMEM),
            pl.BlockSpec(memory_space=pltpu.SMEM),
            pl.BlockSpec(memory_space=pltpu.SMEM),
            pl.BlockSpec(memory_space=pltpu.SMEM),
        ],
        out_specs=pl.BlockSpec((BR, N_NODES), lambda i: (i, 0)),
    )(af1, A2, Wm1, bm1, Wm2, bm2)


def _sinkhorn(cost):
    n = N_NODES
    inv = 1.0 / n

    NB = 16
    B = n // NB

    def body(c_r, plan_r, soft_r, u_r):
        def stage(b, t):
            sl = pl.ds(b * B, B)
            plan_r[sl, :] = jnp.exp(c_r[sl, :] * -10.0)
            return t

        lax.fori_loop(0, NB, stage, 0)

        v0 = jnp.full((1, n), inv, jnp.float32)

        def step(_, v):
            def rowblk(b, t):
                sl = pl.ds(b * B, B)
                Kv = jnp.sum(plan_r[sl, :] * v, axis=1, keepdims=True)
                u_r[sl, :] = inv / (Kv + 1e-9)
                return t

            lax.fori_loop(0, NB, rowblk, 0)

            def colblk(b, ku):
                sl = pl.ds(b * B, B)
                return ku + jnp.sum(plan_r[sl, :] * u_r[sl, :],
                                    axis=0, keepdims=True)

            Ku = lax.fori_loop(0, NB, colblk, jnp.zeros((1, n), jnp.float32))
            return inv / (Ku + 1e-9)

        v = lax.fori_loop(0, 10, step, v0)

        def finish(b, s):
            sl = pl.ds(b * B, B)
            p = u_r[sl, :] * plan_r[sl, :] * v
            plan_r[sl, :] = p
            return s + jnp.sum(p * c_r[sl, :]).reshape(1, 1)

        soft_r[...] = lax.fori_loop(0, NB, finish,
                                    jnp.zeros((1, 1), jnp.float32))

    return pl.pallas_call(
        body,
        out_shape=(
            jax.ShapeDtypeStruct((n, n), jnp.float32),
            jax.ShapeDtypeStruct((1, 1), jnp.float32),
        ),
        scratch_shapes=[pltpu.VMEM((n, 1), jnp.float32)],
    )(cost)


def _head(af1, af2, soft, Wa, Wt, WbT, btT, fW1, fb1, fW2, fb2, fW3, fb3,
          fWs, fbs, avg_v):
    def body(a1_r, a2_r, soft_r, Wa_r, Wt_r, WbT_r, btT_r, fW1_r, fb1_r,
             fW2_r, fb2_r, fW3_r, fb3_r, fWs_r, fbs_r, av_r,
             score_r, ged_r):
        Wav = Wa_r[...]

        def attention(x):
            t = jnp.dot(x, Wav, preferred_element_type=jnp.float32)
            gc = jnp.tanh(jnp.mean(t, axis=0, keepdims=True))
            s = jax.nn.sigmoid(
                lax.dot_general(x, gc, (((1,), (1,)), ((), ())),
                                preferred_element_type=jnp.float32))
            return lax.dot_general(x, s, (((0,), (0,)), ((), ())),
                                   preferred_element_type=jnp.float32)

        e1 = attention(a1_r[...])
        e2 = attention(a2_r[...])
        M = jnp.sum(e1[:, :, None] * Wt_r[...], axis=0)
        zT = lax.dot_general(e2, M, (((0,), (0,)), ((), ())),
                             preferred_element_type=jnp.float32)
        blockT = (lax.dot_general(e1, WbT_r[:e1.shape[0]],
                                  (((0,), (0,)), ((), ())),
                                  preferred_element_type=jnp.float32)
                  + lax.dot_general(e2, WbT_r[e1.shape[0]:],
                                    (((0,), (0,)), ((), ())),
                                    preferred_element_type=jnp.float32))
        s = jnp.maximum(zT + blockT + btT_r[...], 0.0)
        s = jnp.maximum(jnp.dot(s, fW1_r[...],
                                preferred_element_type=jnp.float32)
                        + fb1_r[...], 0.0)
        s = jnp.maximum(jnp.dot(s, fW2_r[...],
                                preferred_element_type=jnp.float32)
                        + fb2_r[...], 0.0)
        s = jnp.maximum(jnp.dot(s, fW3_r[...],
                                preferred_element_type=jnp.float32)
                        + fb3_r[...], 0.0)
        bias = (jnp.dot(s, fWs_r[...], preferred_element_type=jnp.float32)
                + fbs_r[...])
        score = jax.nn.sigmoid(soft_r[...] + bias)
        score_r[...] = score
        ged_r[...] = -jnp.log(score) * av_r[...]

    return pl.pallas_call(
        body,
        out_shape=(
            jax.ShapeDtypeStruct((1, 1), jnp.float32),
            jax.ShapeDtypeStruct((1, 1), jnp.float32),
        ),
    )(af1, af2, soft, Wa, Wt, WbT, btT, fW1, fb1, fW2, fb2, fW3, fb3,
      fWs, fbs, avg_v)


def kernel(features_1, features_2, edge_index_1, edge_index_2, avg_v, params):
    src = jnp.concatenate([edge_index_1[0], edge_index_2[0] + N_NODES])
    dst = jnp.concatenate([edge_index_1[1], edge_index_2[1] + N_NODES])
    x = jnp.concatenate([features_1, features_2], axis=0)

    x0 = x
    feats = [x0]
    zeros256 = jnp.zeros((R, 256), jnp.float32)
    for name in ("c1", "c2", "c3"):
        p = params[name]
        F = x.shape[1]
        agg = _sc_segment_sum(x, src, dst, zeros256[:, :F])
        scale = (1.0 + p["eps"]).reshape(1, 1)
        f, x = _gin_dense(x, agg[0], agg[1], scale,
                          p["W1"], p["b1"].reshape(1, -1),
                          p["W2"], p["b2"].reshape(1, -1),
                          p["g"].reshape(1, -1), p["be"].reshape(1, -1))
        feats.append(f)

    m = params["mlp"]
    af = _mlp(feats[0], feats[1], feats[2], feats[3],
              m["W1"], m["b1"].reshape(1, -1),
              m["W2"], m["b2"].reshape(1, -1),
              m["W3"], m["b3"].reshape(1, -1))
    af1, af2 = af[:N_NODES], af[N_NODES:]

    c = params["cost"]
    KH, F3, _ = c["Wc"].shape
    Wcflat = c["Wc"].transpose(2, 0, 1).reshape(F3, KH * F3)
    A2 = _a2(af2, Wcflat)
    cost = _cost(af1, A2, c["Wm1"], c["bm1"], c["Wm2"][:, 0], c["bm2"])

    plan, soft = _sinkhorn(cost)

    t = params["tn"]
    fc = params["fc"]
    TN = t["Wt"].shape[2]
    score, ged = _head(
        af1, af2, soft, params["att"]["Wa"],
        t["Wt"], t["Wb"].T, t["bt"].reshape(1, TN),
        fc["W1"], fc["b1"].reshape(1, -1),
        fc["W2"], fc["b2"].reshape(1, -1),
        fc["W3"], fc["b3"].reshape(1, -1),
        fc["Ws"], fc["bs"].reshape(1, 1),
        avg_v.reshape(1, 1))
    return score.reshape(-1), ged.reshape(-1), plan
